# baseline (device time: 80468 ns/iter reference)
import jax
import jax.numpy as jnp
from jax import lax
from jax.experimental import pallas as pl
from jax.experimental.pallas import tpu as pltpu

N_DEV = 4
BQ = 2
SQ = 512
SKV = 512
HG = 8
DH = 64
DM = 768
DG = HG * DH
NC = 2
HH = HG // NC
HDG = HH * DH
NSTEP = (N_DEV - 1) * NC


def kernel(x, Wq, K_ext, V_ext, Wo):
    bf16 = jnp.bfloat16
    my = lax.axis_index("i")

    Wqb = Wq.astype(bf16).reshape(DM, NC, HDG).transpose(1, 0, 2)
    Wob = Wo.astype(bf16)

    def regroup(a):
        a = lax.dynamic_slice_in_dim(a, my * BQ, BQ, axis=0)
        a = a.reshape(BQ, SKV, N_DEV, HG, DH).transpose(2, 0, 3, 1, 4)
        return a.astype(bf16)

    Kg = regroup(K_ext)
    Vg = regroup(V_ext)

    def body(x_ref, wq_ref, k_ref, v_ref, wo_ref, out_ref,
             wq_buf, wo_buf, wq_send, wq_recv, wo_send, wo_recv):
        my_pos = lax.axis_index("i")
        left = lax.rem(my_pos + N_DEV - 1, N_DEV)
        right = lax.rem(my_pos + 1, N_DEV)

        barrier = pltpu.get_barrier_semaphore()
        pl.semaphore_signal(barrier, inc=1, device_id=(left,),
                            device_id_type=pl.DeviceIdType.MESH)
        pl.semaphore_signal(barrier, inc=1, device_id=(right,),
                            device_id_type=pl.DeviceIdType.MESH)
        pl.semaphore_wait(barrier, 2)

        qi = lax.broadcasted_iota(jnp.int32, (SQ, SKV), 0) // 64
        kj = lax.broadcasted_iota(jnp.int32, (SQ, SKV), 1) // 64
        mask = (kj % 4) == (qi % 4)

        def attn(slot, c, wq_half):
            origin = lax.rem(my_pos - slot + N_DEV, N_DEV)
            ctxs = []
            for b in range(BQ):
                q = jnp.dot(x_ref[b].astype(bf16), wq_half,
                            preferred_element_type=jnp.float32).astype(bf16)
                qh = jnp.transpose(q.reshape(SQ, HH, DH), (1, 0, 2))
                kg = k_ref[origin, b, c * HH:(c + 1) * HH]
                vg = v_ref[origin, b, c * HH:(c + 1) * HH]
                s = lax.dot_general(
                    qh, kg, (((2,), (2,)), ((0,), (0,))),
                    preferred_element_type=jnp.float32) * 0.125
                s = jnp.where(mask[None], s, -1e9)
                w = jnp.exp(s)
                w = (w / jnp.sum(w, axis=-1, keepdims=True)).astype(bf16)
                ctx = lax.dot_general(
                    w, vg, (((2,), (1,)), ((0,), (0,))),
                    preferred_element_type=jnp.float32).astype(bf16)
                ctxs.append(jnp.transpose(ctx, (1, 0, 2)).reshape(SQ, HDG))
            return ctxs

        def proj(slot, c, ctxs, wo_half):
            for b in range(BQ):
                part = jnp.dot(ctxs[b], wo_half,
                               preferred_element_type=jnp.float32)
                if slot == 0 and c == 0:
                    out_ref[b, :, :] = part
                else:
                    out_ref[b, :, :] = out_ref[b, :, :] + part

        sends = []

        def send(t, src, buf, send_sems, recv_sems):
            r = pltpu.make_async_remote_copy(
                src_ref=src, dst_ref=buf.at[t],
                send_sem=send_sems.at[t], recv_sem=recv_sems.at[t],
                device_id=(right,), device_id_type=pl.DeviceIdType.MESH)
            r.start()
            sends.append(r)

        def wait_recv(t, buf, send_sems, recv_sems):
            r = pltpu.make_async_remote_copy(
                src_ref=buf.at[t], dst_ref=buf.at[t],
                send_sem=send_sems.at[t], recv_sem=recv_sems.at[t],
                device_id=(left,), device_id_type=pl.DeviceIdType.MESH)
            r.wait_recv()

        for c in range(NC):
            send(c, wq_ref.at[c], wq_buf, wq_send, wq_recv)
            send(c, wo_ref.at[c * HDG:(c + 1) * HDG, :], wo_buf,
                 wo_send, wo_recv)
        for c in range(NC):
            ctxs = attn(0, c, wq_ref[c])
            proj(0, c, ctxs, wo_ref[c * HDG:(c + 1) * HDG, :])

        for h in range(N_DEV - 1):
            for c in range(NC):
                t = NC * h + c
                wait_recv(t, wq_buf, wq_send, wq_recv)
                if h < N_DEV - 2:
                    send(t + NC, wq_buf.at[t], wq_buf, wq_send, wq_recv)
                ctxs = attn(h + 1, c, wq_buf[t])
                wait_recv(t, wo_buf, wo_send, wo_recv)
                if h < N_DEV - 2:
                    send(t + NC, wo_buf.at[t], wo_buf, wo_send, wo_recv)
                proj(h + 1, c, ctxs, wo_buf[t])

        for r in sends:
            r.wait_send()

    out = pl.pallas_call(
        body,
        out_shape=jax.ShapeDtypeStruct((BQ, SQ, DM), jnp.float32),
        in_specs=[pl.BlockSpec(memory_space=pltpu.VMEM)] * 5,
        out_specs=pl.BlockSpec(memory_space=pltpu.VMEM),
        scratch_shapes=[
            pltpu.VMEM((NSTEP, DM, HDG), jnp.bfloat16),
            pltpu.VMEM((NSTEP, HDG, DM), jnp.bfloat16),
            pltpu.SemaphoreType.DMA((NSTEP,)),
            pltpu.SemaphoreType.DMA((NSTEP,)),
            pltpu.SemaphoreType.DMA((NSTEP,)),
            pltpu.SemaphoreType.DMA((NSTEP,)),
        ],
        compiler_params=pltpu.CompilerParams(collective_id=0),
    )(x, Wqb, Kg, Vg, Wob)
    return out


# device time: 79234 ns/iter; 1.0156x vs baseline; 1.0156x over previous
import jax
import jax.numpy as jnp
from jax import lax
from jax.experimental import pallas as pl
from jax.experimental.pallas import tpu as pltpu

N_DEV = 4
BQ = 2
SQ = 512
SKV = 512
HG = 8
DH = 64
DM = 768
DG = HG * DH


def kernel(x, Wq, K_ext, V_ext, Wo):
    bf16 = jnp.bfloat16
    my = lax.axis_index("i")

    Wqb = Wq.astype(bf16)
    Wob = Wo.astype(bf16)

    def regroup(a):
        a = lax.dynamic_slice_in_dim(a, my * BQ, BQ, axis=0)
        a = a.reshape(BQ, SKV, N_DEV, HG, DH).transpose(2, 0, 3, 1, 4)
        return a.astype(bf16)

    Kg = regroup(K_ext)
    Vg = regroup(V_ext)

    def body(x_ref, wq_ref, k_ref, v_ref, wo_ref, out_ref,
             wq_buf, wo_buf, wq_send, wq_recv, wo_send, wo_recv):
        my_pos = lax.axis_index("i")
        left = lax.rem(my_pos + N_DEV - 1, N_DEV)
        right = lax.rem(my_pos + 1, N_DEV)

        barrier = pltpu.get_barrier_semaphore()
        pl.semaphore_signal(barrier, inc=1, device_id=(left,),
                            device_id_type=pl.DeviceIdType.MESH)
        pl.semaphore_signal(barrier, inc=1, device_id=(right,),
                            device_id_type=pl.DeviceIdType.MESH)
        pl.semaphore_wait(barrier, 2)

        qi = lax.broadcasted_iota(jnp.int32, (SQ, SKV), 0) // 64
        kj = lax.broadcasted_iota(jnp.int32, (SQ, SKV), 1) // 64
        mask = (kj % 4) == (qi % 4)

        def attn(slot, wq):
            origin = lax.rem(my_pos - slot + N_DEV, N_DEV)
            ctxs = []
            for b in range(BQ):
                q = jnp.dot(x_ref[b].astype(bf16), wq,
                            preferred_element_type=jnp.float32).astype(bf16)
                qh = jnp.transpose(q.reshape(SQ, HG, DH), (1, 0, 2))
                kg = k_ref[origin, b]
                vg = v_ref[origin, b]
                s = lax.dot_general(
                    qh, kg, (((2,), (2,)), ((0,), (0,))),
                    preferred_element_type=jnp.float32) * 0.125
                s = jnp.where(mask[None], s, -1e9)
                w = jnp.exp(s)
                w = (w / jnp.sum(w, axis=-1, keepdims=True)).astype(bf16)
                ctx = lax.dot_general(
                    w, vg, (((2,), (1,)), ((0,), (0,))),
                    preferred_element_type=jnp.float32).astype(bf16)
                ctxs.append(jnp.transpose(ctx, (1, 0, 2)).reshape(SQ, DG))
            return ctxs

        def proj(slot, ctxs, wo):
            for b in range(BQ):
                part = jnp.dot(ctxs[b], wo, preferred_element_type=jnp.float32)
                if slot == 0:
                    out_ref[b, :, :] = part
                else:
                    out_ref[b, :, :] = out_ref[b, :, :] + part

        sends = []

        def send(t, src, buf, send_sems, recv_sems):
            r = pltpu.make_async_remote_copy(
                src_ref=src, dst_ref=buf.at[t],
                send_sem=send_sems.at[t], recv_sem=recv_sems.at[t],
                device_id=(right,), device_id_type=pl.DeviceIdType.MESH)
            r.start()
            sends.append(r)

        def wait_recv(t, buf, send_sems, recv_sems):
            r = pltpu.make_async_remote_copy(
                src_ref=buf.at[t], dst_ref=buf.at[t],
                send_sem=send_sems.at[t], recv_sem=recv_sems.at[t],
                device_id=(left,), device_id_type=pl.DeviceIdType.MESH)
            r.wait_recv()

        send(0, wq_ref, wq_buf, wq_send, wq_recv)
        send(0, wo_ref, wo_buf, wo_send, wo_recv)
        ctxs = attn(0, wq_ref[...])
        proj(0, ctxs, wo_ref[...])

        for t in range(N_DEV - 1):
            wait_recv(t, wq_buf, wq_send, wq_recv)
            if t < N_DEV - 2:
                send(t + 1, wq_buf.at[t], wq_buf, wq_send, wq_recv)
            ctxs = attn(t + 1, wq_buf[t])
            wait_recv(t, wo_buf, wo_send, wo_recv)
            if t < N_DEV - 2:
                send(t + 1, wo_buf.at[t], wo_buf, wo_send, wo_recv)
            proj(t + 1, ctxs, wo_buf[t])

        for r in sends:
            r.wait_send()

    out = pl.pallas_call(
        body,
        out_shape=jax.ShapeDtypeStruct((BQ, SQ, DM), jnp.float32),
        in_specs=[pl.BlockSpec(memory_space=pltpu.VMEM)] * 5,
        out_specs=pl.BlockSpec(memory_space=pltpu.VMEM),
        scratch_shapes=[
            pltpu.VMEM((N_DEV - 1, DM, DG), jnp.bfloat16),
            pltpu.VMEM((N_DEV - 1, DG, DM), jnp.bfloat16),
            pltpu.SemaphoreType.DMA((N_DEV - 1,)),
            pltpu.SemaphoreType.DMA((N_DEV - 1,)),
            pltpu.SemaphoreType.DMA((N_DEV - 1,)),
            pltpu.SemaphoreType.DMA((N_DEV - 1,)),
        ],
        compiler_params=pltpu.CompilerParams(collective_id=0),
    )(x, Wqb, Kg, Vg, Wob)
    return out
